# Initial kernel scaffold; baseline (speedup 1.0000x reference)
import functools

import jax
import jax.numpy as jnp
from jax import lax
from jax.experimental import pallas as pl
from jax.experimental.pallas import tpu as pltpu
from jax.experimental.pallas import tpu_sc as plsc

N = 10000
E = 320000
IN = 128
OUT = 128
R = 8
T = 4
J = R + T

NC = 2
NS = 16
NW = NC * NS

CH = 128
EPW = E // NW
NCH = -(-EPW // CH)
EPW_PAD = NCH * CH

NP = 10240
ROWS_PER_TILE = NP // NS
INIT_CH = ROWS_PER_TILE // CH

BN = 1000
NBLK = N // BN


def _mm_body(feat_ref, w_ref, b_ref, out_ref):
    out_ref[0] = (
        jnp.dot(feat_ref[...], w_ref[0], preferred_element_type=jnp.float32)
        + b_ref[0]
    )


def _add_body(a_ref, b_ref, out_ref):
    out_ref[...] = a_ref[0] + b_ref[0]


def _sc_body(table_hbm, src_hbm, et_hbm, dst_hbm, att_hbm, nt_hbm, parts_hbm,
             acc, idx_v, dst_v, att_v, rows_v, src_v, et_v, gsem):
    c = lax.axis_index("c")
    s = lax.axis_index("s")
    row_base = s * ROWS_PER_TILE

    @pl.when(c == 0)
    def _init_loop():
        def ibody(k, _):
            gb = row_base + k * CH
            pltpu.sync_copy(nt_hbm.at[pl.ds(gb, CH)], src_v)
            for q in range(CH // 16):
                nt16 = src_v[pl.ds(q * 16, 16)]
                rows16 = gb + q * 16 + lax.iota(jnp.int32, 16)
                idx_v[0, pl.ds(q * 16, 16)] = (nt16 + R) * N + rows16
            pltpu.async_copy(table_hbm.at[idx_v.at[0]], rows_v.at[0], gsem).wait()
            pltpu.sync_copy(rows_v.at[0], acc.at[pl.ds(gb, CH)])
            return 0

        lax.fori_loop(0, INIT_CH, ibody, 0)

    @pl.when(c == 1)
    def _init_zero():
        def zbody(r, _):
            for q in range(OUT // 16):
                rows_v[0, r, pl.ds(q * 16, 16)] = jnp.zeros((16,), jnp.float32)
            return 0

        lax.fori_loop(0, CH, zbody, 0)
        for k in range(INIT_CH):
            pltpu.sync_copy(rows_v.at[0], acc.at[pl.ds(row_base + k * CH, CH)])

    plsc.subcore_barrier()

    wid = s * NC + c
    ebase = wid * EPW_PAD

    def chunk(i, _):
        off = ebase + i * CH
        pltpu.sync_copy(src_hbm.at[pl.ds(off, CH)], src_v)
        pltpu.sync_copy(et_hbm.at[pl.ds(off, CH)], et_v)
        pltpu.sync_copy(dst_hbm.at[pl.ds(off, CH)], dst_v.at[0])
        pltpu.sync_copy(att_hbm.at[pl.ds(off, CH)], att_v)
        for q in range(CH // 16):
            sl = pl.ds(q * 16, 16)
            idx_v[0, sl] = et_v[sl] * N + src_v[sl]
        pltpu.async_copy(table_hbm.at[idx_v.at[0]], rows_v.at[0], gsem).wait()

        def sbody(e, _):
            spl = plsc.load_gather(att_v, [jnp.full((16,), e, jnp.int32)])
            for q in range(OUT // 16):
                sl = pl.ds(q * 16, 16)
                rows_v[0, e, sl] = rows_v[0, e, sl] * spl
            return 0

        lax.fori_loop(0, CH, sbody, 0)
        pltpu.sync_copy(rows_v.at[0], acc.at[dst_v.at[0]], add=True)
        return 0

    lax.fori_loop(0, NCH, chunk, 0)

    plsc.subcore_barrier()
    for k in range(INIT_CH):
        rb = row_base + k * CH
        pltpu.sync_copy(acc.at[pl.ds(rb, CH)], rows_v.at[0])
        pltpu.sync_copy(rows_v.at[0], parts_hbm.at[c, pl.ds(rb, CH)])


_sc_kernel = functools.partial(
    pl.kernel,
    out_type=jax.ShapeDtypeStruct((NC, NP, OUT), jnp.float32),
    mesh=plsc.VectorSubcoreMesh(
        core_axis_name="c", subcore_axis_name="s", num_cores=NC, num_subcores=NS
    ),
    scratch_types=[
        pltpu.VMEM_SHARED((NP, OUT), jnp.float32),
        pltpu.VMEM((1, CH), jnp.int32),
        pltpu.VMEM((1, CH), jnp.int32),
        pltpu.VMEM((CH,), jnp.float32),
        pltpu.VMEM((1, CH, OUT), jnp.float32),
        pltpu.VMEM((CH,), jnp.int32),
        pltpu.VMEM((CH,), jnp.int32),
        pltpu.SemaphoreType.DMA,
    ],
)(_sc_body)


def _pad_per_worker(x, fill):
    x2 = x.reshape(NW, EPW)
    return jnp.pad(x2, ((0, 0), (0, EPW_PAD - EPW)), constant_values=fill).reshape(-1)


def kernel(feat, edge_index, etypes, ntypes, attention, weight, m_bias,
           loop_weight, h_bias):
    w_all = jnp.concatenate([weight, loop_weight], axis=0)
    b_all = jnp.concatenate([m_bias, h_bias[:, 0, :]], axis=0)

    table = pl.pallas_call(
        _mm_body,
        grid=(J, NBLK),
        in_specs=[
            pl.BlockSpec((BN, IN), lambda j, b: (b, 0)),
            pl.BlockSpec((1, IN, OUT), lambda j, b: (j, 0, 0)),
            pl.BlockSpec((1, OUT), lambda j, b: (j, 0)),
        ],
        out_specs=pl.BlockSpec((1, BN, OUT), lambda j, b: (j, b, 0)),
        out_shape=jax.ShapeDtypeStruct((J, N, OUT), jnp.float32),
    )(feat, w_all, b_all)
    table_flat = table.reshape(J * N, OUT)

    srcp = _pad_per_worker(edge_index[0], 0)
    etp = _pad_per_worker(etypes, 0)
    dstp = _pad_per_worker(edge_index[1], 0)
    attp = _pad_per_worker(attention.reshape(E), 0.0)
    ntp = jnp.pad(ntypes, (0, NP - N))

    parts = _sc_kernel(table_flat, srcp, etp, dstp, attp, ntp)

    out = pl.pallas_call(
        _add_body,
        grid=(NBLK,),
        in_specs=[
            pl.BlockSpec((1, BN, OUT), lambda b: (0, b, 0)),
            pl.BlockSpec((1, BN, OUT), lambda b: (1, b, 0)),
        ],
        out_specs=pl.BlockSpec((BN, OUT), lambda b: (b, 0)),
        out_shape=jax.ShapeDtypeStruct((N, OUT), jnp.float32),
    )(parts, parts)
    return out.reshape(N, 1, OUT)


# speedup vs baseline: 11.5575x; 11.5575x over previous
import functools

import jax
import jax.numpy as jnp
from jax import lax
from jax.experimental import pallas as pl
from jax.experimental.pallas import tpu as pltpu
from jax.experimental.pallas import tpu_sc as plsc

N = 10000
E = 320000
IN = 128
OUT = 128
R = 8
T = 4
J = R + T

NC = 2
NS = 16
NW = NC * NS

CH = 128
EPW = E // NW
NCH = -(-EPW // CH)
EPW_PAD = NCH * CH

NP = 10240
ROWS_PER_TILE = NP // NS
INIT_CH = ROWS_PER_TILE // CH

BN = 1000
NBLK = N // BN


def _mm_body(feat_ref, w_ref, b_ref, out_ref):
    out_ref[0] = (
        jnp.dot(feat_ref[...], w_ref[0], preferred_element_type=jnp.float32)
        + b_ref[0, 0]
    )


def _add_body(a_ref, b_ref, out_ref):
    out_ref[...] = a_ref[0] + b_ref[0]


def _sc_body(table_hbm, src_hbm, et_hbm, dst_hbm, att_hbm, nt_hbm, parts_hbm,
             acc, idx_v, dst_v, att_v, rows_v, src_v, et_v, gsem):
    c = lax.axis_index("c")
    s = lax.axis_index("s")
    row_base = s * ROWS_PER_TILE

    @pl.when(c == 0)
    def _init_loop():
        def ibody(k, _):
            gb = row_base + k * CH
            pltpu.sync_copy(nt_hbm.at[pl.ds(gb, CH)], src_v)
            for q in range(CH // 16):
                nt16 = src_v[pl.ds(q * 16, 16)]
                rows16 = gb + q * 16 + lax.iota(jnp.int32, 16)
                idx_v[0, pl.ds(q * 16, 16)] = (nt16 + R) * N + rows16
            pltpu.async_copy(table_hbm.at[idx_v.at[0]], rows_v.at[0], gsem).wait()
            pltpu.sync_copy(rows_v.at[0], acc.at[pl.ds(gb, CH)])
            return 0

        lax.fori_loop(0, INIT_CH, ibody, 0)

    @pl.when(c == 1)
    def _init_zero():
        def zbody(r, _):
            for q in range(OUT // 16):
                rows_v[0, r, pl.ds(q * 16, 16)] = jnp.zeros((16,), jnp.float32)
            return 0

        lax.fori_loop(0, CH, zbody, 0)
        for k in range(INIT_CH):
            pltpu.sync_copy(rows_v.at[0], acc.at[pl.ds(row_base + k * CH, CH)])

    plsc.subcore_barrier()

    wid = s * NC + c
    ebase = wid * EPW_PAD

    def chunk(i, _):
        off = ebase + i * CH
        pltpu.sync_copy(src_hbm.at[pl.ds(off, CH)], src_v)
        pltpu.sync_copy(et_hbm.at[pl.ds(off, CH)], et_v)
        pltpu.sync_copy(dst_hbm.at[pl.ds(off, CH)], dst_v.at[0])
        pltpu.sync_copy(att_hbm.at[pl.ds(off, CH)], att_v)
        for q in range(CH // 16):
            sl = pl.ds(q * 16, 16)
            idx_v[0, sl] = et_v[sl] * N + src_v[sl]
        pltpu.async_copy(table_hbm.at[idx_v.at[0]], rows_v.at[0], gsem).wait()

        def sbody(g, _):
            att16 = att_v[pl.ds(g * 16, 16)]
            for j in range(16):
                e = g * 16 + j
                spl = jnp.full((16,), att16[j])
                for q in range(OUT // 16):
                    sl = pl.ds(q * 16, 16)
                    rows_v[0, e, sl] = rows_v[0, e, sl] * spl
            return 0

        lax.fori_loop(0, CH // 16, sbody, 0)
        pltpu.sync_copy(rows_v.at[0], acc.at[dst_v.at[0]], add=True)
        return 0

    lax.fori_loop(0, NCH, chunk, 0)

    plsc.subcore_barrier()
    for k in range(INIT_CH):
        rb = row_base + k * CH
        pltpu.sync_copy(acc.at[pl.ds(rb, CH)], rows_v.at[0])
        pltpu.sync_copy(rows_v.at[0], parts_hbm.at[c, pl.ds(rb, CH)])


_sc_kernel = functools.partial(
    pl.kernel,
    out_type=jax.ShapeDtypeStruct((NC, NP, OUT), jnp.float32),
    mesh=plsc.VectorSubcoreMesh(
        core_axis_name="c", subcore_axis_name="s", num_cores=NC, num_subcores=NS
    ),
    scratch_types=[
        pltpu.VMEM_SHARED((NP, OUT), jnp.float32),
        pltpu.VMEM((1, CH), jnp.int32),
        pltpu.VMEM((1, CH), jnp.int32),
        pltpu.VMEM((CH,), jnp.float32),
        pltpu.VMEM((1, CH, OUT), jnp.float32),
        pltpu.VMEM((CH,), jnp.int32),
        pltpu.VMEM((CH,), jnp.int32),
        pltpu.SemaphoreType.DMA,
    ],
)(_sc_body)


def _pad_per_worker(x, fill):
    x2 = x.reshape(NW, EPW)
    return jnp.pad(x2, ((0, 0), (0, EPW_PAD - EPW)), constant_values=fill).reshape(-1)


def kernel(feat, edge_index, etypes, ntypes, attention, weight, m_bias,
           loop_weight, h_bias):
    w_all = jnp.concatenate([weight, loop_weight], axis=0)
    b_all = jnp.concatenate([m_bias, h_bias[:, 0, :]], axis=0)
    b_all = b_all.reshape(J, 1, OUT)

    table = pl.pallas_call(
        _mm_body,
        grid=(J, NBLK),
        in_specs=[
            pl.BlockSpec((BN, IN), lambda j, b: (b, 0)),
            pl.BlockSpec((1, IN, OUT), lambda j, b: (j, 0, 0)),
            pl.BlockSpec((1, 1, OUT), lambda j, b: (j, 0, 0)),
        ],
        out_specs=pl.BlockSpec((1, BN, OUT), lambda j, b: (j, b, 0)),
        out_shape=jax.ShapeDtypeStruct((J, N, OUT), jnp.float32),
    )(feat, w_all, b_all)
    table_flat = table.reshape(J * N, OUT)

    srcp = _pad_per_worker(edge_index[0], 0)
    etp = _pad_per_worker(etypes, 0)
    dstp = _pad_per_worker(edge_index[1], 0)
    attp = _pad_per_worker(attention.reshape(E), 0.0)
    ntp = jnp.pad(ntypes, (0, NP - N))

    parts = _sc_kernel(table_flat, srcp, etp, dstp, attp, ntp)

    out = pl.pallas_call(
        _add_body,
        grid=(NBLK,),
        in_specs=[
            pl.BlockSpec((1, BN, OUT), lambda b: (0, b, 0)),
            pl.BlockSpec((1, BN, OUT), lambda b: (1, b, 0)),
        ],
        out_specs=pl.BlockSpec((BN, OUT), lambda b: (b, 0)),
        out_shape=jax.ShapeDtypeStruct((N, OUT), jnp.float32),
    )(parts, parts)
    return out.reshape(N, 1, OUT)


# speedup vs baseline: 12.7333x; 1.1017x over previous
import functools

import jax
import jax.numpy as jnp
from jax import lax
from jax.experimental import pallas as pl
from jax.experimental.pallas import tpu as pltpu
from jax.experimental.pallas import tpu_sc as plsc

N = 10000
E = 320000
IN = 128
OUT = 128
R = 8
T = 4
J = R + T

NC = 2
NS = 16
NW = NC * NS

CH = 128
EPW = E // NW
NCH = 80
EPW_PAD = NCH * CH

NP = 10240
ROWS_PER_TILE = NP // NS
INIT_CH = ROWS_PER_TILE // CH

BN = 1000
NBLK = N // BN


def _mm_body(feat_ref, w_ref, b_ref, out_ref):
    out_ref[0] = (
        jnp.dot(feat_ref[...], w_ref[0], preferred_element_type=jnp.float32)
        + b_ref[0, 0]
    )


def _add_body(a_ref, b_ref, out_ref):
    out_ref[...] = a_ref[0] + b_ref[0]


def _idx_body(s_ref, e_ref, out_ref):
    out_ref[...] = e_ref[...] * N + s_ref[...]


def _sc_body(table_hbm, idx_hbm, dst_hbm, att_hbm, nt_hbm, parts_hbm,
             acc, idx_all, dst_v, att_v, rows_v, nt_v,
             gsem0, gsem1, msem):
    c = lax.axis_index("c")
    s = lax.axis_index("s")
    row_base = s * ROWS_PER_TILE
    wid = s * NC + c
    ebase = wid * EPW_PAD

    meta = pltpu.async_copy(idx_hbm.at[pl.ds(ebase, EPW_PAD)], idx_all, msem)

    @pl.when(c == 0)
    def _init_loop():
        def ibody(k, _):
            gb = row_base + k * CH
            pltpu.sync_copy(nt_hbm.at[pl.ds(gb, CH)], nt_v.at[0])
            for q in range(CH // 16):
                nt16 = nt_v[0, pl.ds(q * 16, 16)]
                rows16 = gb + q * 16 + lax.iota(jnp.int32, 16)
                nt_v[1, pl.ds(q * 16, 16)] = (nt16 + R) * N + rows16
            pltpu.async_copy(table_hbm.at[nt_v.at[1]], rows_v.at[0], gsem0).wait()
            pltpu.sync_copy(rows_v.at[0], acc.at[pl.ds(gb, CH)])
            return 0

        lax.fori_loop(0, INIT_CH, ibody, 0)

    @pl.when(c == 1)
    def _init_zero():
        def zbody(r, _):
            for q in range(OUT // 16):
                rows_v[0, r, pl.ds(q * 16, 16)] = jnp.zeros((16,), jnp.float32)
            return 0

        lax.fori_loop(0, CH, zbody, 0)
        for k in range(INIT_CH):
            pltpu.sync_copy(rows_v.at[0], acc.at[pl.ds(row_base + k * CH, CH)])

    meta.wait()

    def start_chunk(ci, slot, sem):
        pltpu.async_copy(
            table_hbm.at[idx_all.at[pl.ds(ci * CH, CH)]], rows_v.at[slot], sem
        )
        pltpu.async_copy(att_hbm.at[pl.ds(ebase + ci * CH, CH)],
                         att_v.at[slot], sem)
        pltpu.async_copy(dst_hbm.at[wid * NCH + ci], dst_v.at[slot], sem)

    def wait_chunk(slot, sem):
        pltpu.make_async_copy(
            table_hbm.at[idx_all.at[pl.ds(0, CH)]], rows_v.at[slot], sem
        ).wait()
        pltpu.make_async_copy(att_hbm.at[pl.ds(0, CH)], att_v.at[slot],
                              sem).wait()
        pltpu.make_async_copy(dst_hbm.at[0], dst_v.at[slot], sem).wait()

    def scale_scatter(ci, slot):
        def sbody(g, _):
            att16 = att_v[slot, pl.ds(g * 16, 16)]
            for j in range(16):
                e = g * 16 + j
                spl = jnp.full((16,), att16[j])
                for q in range(OUT // 16):
                    sl = pl.ds(q * 16, 16)
                    rows_v[slot, e, sl] = rows_v[slot, e, sl] * spl
            return 0

        lax.fori_loop(0, CH // 16, sbody, 0)
        pltpu.sync_copy(rows_v.at[slot], acc.at[dst_v.at[slot]], add=True)

    start_chunk(0, 0, gsem0)
    start_chunk(1, 1, gsem1)

    plsc.subcore_barrier()

    def pair(t, _):
        ca = 2 * t
        wait_chunk(0, gsem0)
        scale_scatter(ca, 0)

        @pl.when(ca + 2 < NCH)
        def _():
            start_chunk(ca + 2, 0, gsem0)

        wait_chunk(1, gsem1)
        scale_scatter(ca + 1, 1)

        @pl.when(ca + 3 < NCH)
        def _():
            start_chunk(ca + 3, 1, gsem1)

        return 0

    lax.fori_loop(0, NCH // 2, pair, 0)

    plsc.subcore_barrier()
    for k in range(INIT_CH):
        rb = row_base + k * CH
        pltpu.sync_copy(acc.at[pl.ds(rb, CH)], rows_v.at[0])
        pltpu.sync_copy(rows_v.at[0], parts_hbm.at[c, pl.ds(rb, CH)])


_sc_kernel = functools.partial(
    pl.kernel,
    out_type=jax.ShapeDtypeStruct((NC, NP, OUT), jnp.float32),
    mesh=plsc.VectorSubcoreMesh(
        core_axis_name="c", subcore_axis_name="s", num_cores=NC, num_subcores=NS
    ),
    scratch_types=[
        pltpu.VMEM_SHARED((NP, OUT), jnp.float32),
        pltpu.VMEM((EPW_PAD,), jnp.int32),
        pltpu.VMEM((2, CH), jnp.int32),
        pltpu.VMEM((2, CH), jnp.float32),
        pltpu.VMEM((2, CH, OUT), jnp.float32),
        pltpu.VMEM((2, CH), jnp.int32),
        pltpu.SemaphoreType.DMA,
        pltpu.SemaphoreType.DMA,
        pltpu.SemaphoreType.DMA,
    ],
)(_sc_body)


def _pad_per_worker(x, fill):
    x2 = x.reshape(NW, EPW)
    return jnp.pad(x2, ((0, 0), (0, EPW_PAD - EPW)), constant_values=fill).reshape(-1)


def kernel(feat, edge_index, etypes, ntypes, attention, weight, m_bias,
           loop_weight, h_bias):
    w_all = jnp.concatenate([weight, loop_weight], axis=0)
    b_all = jnp.concatenate([m_bias, h_bias[:, 0, :]], axis=0)
    b_all = b_all.reshape(J, 1, OUT)

    table = pl.pallas_call(
        _mm_body,
        grid=(J, NBLK),
        in_specs=[
            pl.BlockSpec((BN, IN), lambda j, b: (b, 0)),
            pl.BlockSpec((1, IN, OUT), lambda j, b: (j, 0, 0)),
            pl.BlockSpec((1, 1, OUT), lambda j, b: (j, 0, 0)),
        ],
        out_specs=pl.BlockSpec((1, BN, OUT), lambda j, b: (j, b, 0)),
        out_shape=jax.ShapeDtypeStruct((J, N, OUT), jnp.float32),
    )(feat, w_all, b_all)
    table_flat = table.reshape(J * N, OUT)

    srcp = _pad_per_worker(edge_index[0], 0).reshape(NW * NCH, CH)
    etp = _pad_per_worker(etypes, 0).reshape(NW * NCH, CH)
    dstp = _pad_per_worker(edge_index[1], 0).reshape(NW * NCH, CH)
    attp = _pad_per_worker(attention.reshape(E), 0.0)
    ntp = jnp.pad(ntypes, (0, NP - N))

    idxg = pl.pallas_call(
        _idx_body,
        out_shape=jax.ShapeDtypeStruct((NW * NCH, CH), jnp.int32),
    )(srcp, etp).reshape(-1)

    parts = _sc_kernel(table_flat, idxg, dstp, attp, ntp)

    out = pl.pallas_call(
        _add_body,
        grid=(NBLK,),
        in_specs=[
            pl.BlockSpec((1, BN, OUT), lambda b: (0, b, 0)),
            pl.BlockSpec((1, BN, OUT), lambda b: (1, b, 0)),
        ],
        out_specs=pl.BlockSpec((BN, OUT), lambda b: (b, 0)),
        out_shape=jax.ShapeDtypeStruct((N, OUT), jnp.float32),
    )(parts, parts)
    return out.reshape(N, 1, OUT)


# speedup vs baseline: 14.6155x; 1.1478x over previous
import functools

import jax
import jax.numpy as jnp
from jax import lax
from jax.experimental import pallas as pl
from jax.experimental.pallas import tpu as pltpu
from jax.experimental.pallas import tpu_sc as plsc

N = 10000
E = 320000
IN = 128
OUT = 128
R = 8
T = 4
J = R + T

NC = 2
NS = 16
NW = NC * NS

CH = 128
EPW = E // NW
NCH = 80
EPW_PAD = NCH * CH

NP = 10240
ROWS_PER_TILE = NP // NS
INIT_CH = ROWS_PER_TILE // CH

BN = 1000
NBLK = N // BN


def _mm_body(feat_ref, w_ref, b_ref, s_ref, e_ref, table_ref, idx_ref):
    for j in range(J):
        table_ref[j] = (
            jnp.dot(feat_ref[...], w_ref[j], preferred_element_type=jnp.float32)
            + b_ref[j, 0]
        )
    idx_ref[...] = e_ref[...] * N + s_ref[...]


def _add_body(a_ref, b_ref, out_ref):
    out_ref[...] = a_ref[0] + b_ref[0]


def _sc_body(table_hbm, idx_hbm, dst_hbm, att_hbm, nt_hbm, parts_hbm,
             acc, idx_all, dst_v, att_v, rows_v, nt_v,
             gsem0, gsem1, msem):
    c = lax.axis_index("c")
    s = lax.axis_index("s")
    row_base = s * ROWS_PER_TILE
    wid = s * NC + c
    ebase = wid * EPW_PAD

    meta = pltpu.async_copy(idx_hbm.at[pl.ds(ebase, EPW_PAD)], idx_all, msem)

    @pl.when(c == 0)
    def _init_loop():
        def ibody(k, _):
            gb = row_base + k * CH
            pltpu.sync_copy(nt_hbm.at[pl.ds(gb, CH)], nt_v.at[0])
            for q in range(CH // 16):
                nt16 = nt_v[0, pl.ds(q * 16, 16)]
                rows16 = gb + q * 16 + lax.iota(jnp.int32, 16)
                nt_v[1, pl.ds(q * 16, 16)] = (nt16 + R) * N + rows16
            pltpu.async_copy(table_hbm.at[nt_v.at[1]], rows_v.at[0], gsem0).wait()
            pltpu.sync_copy(rows_v.at[0], acc.at[pl.ds(gb, CH)])
            return 0

        lax.fori_loop(0, INIT_CH, ibody, 0)

    @pl.when(c == 1)
    def _init_zero():
        def zbody(r, _):
            for q in range(OUT // 16):
                rows_v[0, r, pl.ds(q * 16, 16)] = jnp.zeros((16,), jnp.float32)
            return 0

        lax.fori_loop(0, CH, zbody, 0)
        for k in range(INIT_CH):
            pltpu.sync_copy(rows_v.at[0], acc.at[pl.ds(row_base + k * CH, CH)])

    meta.wait()

    HC = CH // 2

    def start_chunk(ci, slot, sem):
        pltpu.async_copy(
            table_hbm.at[idx_all.at[pl.ds(ci * CH, HC)]],
            rows_v.at[slot, pl.ds(0, HC)], sem
        )
        pltpu.async_copy(
            table_hbm.at[idx_all.at[pl.ds(ci * CH + HC, HC)]],
            rows_v.at[slot, pl.ds(HC, HC)], sem
        )
        pltpu.async_copy(att_hbm.at[pl.ds(ebase + ci * CH, CH)],
                         att_v.at[slot], sem)
        pltpu.async_copy(dst_hbm.at[wid * NCH + ci], dst_v.at[slot], sem)

    def wait_chunk(slot, sem):
        pltpu.make_async_copy(
            table_hbm.at[idx_all.at[pl.ds(0, CH)]], rows_v.at[slot], sem
        ).wait()
        pltpu.make_async_copy(att_hbm.at[pl.ds(0, CH)], att_v.at[slot],
                              sem).wait()
        pltpu.make_async_copy(dst_hbm.at[0], dst_v.at[slot], sem).wait()

    def scale_scatter(ci, slot):
        def sbody(g, _):
            att16 = att_v[slot, pl.ds(g * 16, 16)]
            for j in range(16):
                e = g * 16 + j
                spl = jnp.full((16,), att16[j])
                for q in range(OUT // 16):
                    sl = pl.ds(q * 16, 16)
                    rows_v[slot, e, sl] = rows_v[slot, e, sl] * spl
            return 0

        lax.fori_loop(0, CH // 16, sbody, 0)
        pltpu.sync_copy(rows_v.at[slot], acc.at[dst_v.at[slot]], add=True)

    start_chunk(0, 0, gsem0)
    start_chunk(1, 1, gsem1)

    plsc.subcore_barrier()

    def pair(t, _):
        ca = 2 * t
        wait_chunk(0, gsem0)
        scale_scatter(ca, 0)

        @pl.when(ca + 2 < NCH)
        def _():
            start_chunk(ca + 2, 0, gsem0)

        wait_chunk(1, gsem1)
        scale_scatter(ca + 1, 1)

        @pl.when(ca + 3 < NCH)
        def _():
            start_chunk(ca + 3, 1, gsem1)

        return 0

    lax.fori_loop(0, NCH // 2, pair, 0)

    plsc.subcore_barrier()
    for k in range(INIT_CH):
        rb = row_base + k * CH
        pltpu.sync_copy(acc.at[pl.ds(rb, CH)], rows_v.at[0])
        pltpu.sync_copy(rows_v.at[0], parts_hbm.at[c, pl.ds(rb, CH)])


_sc_kernel = functools.partial(
    pl.kernel,
    out_type=jax.ShapeDtypeStruct((NC, NP, OUT), jnp.float32),
    mesh=plsc.VectorSubcoreMesh(
        core_axis_name="c", subcore_axis_name="s", num_cores=NC, num_subcores=NS
    ),
    scratch_types=[
        pltpu.VMEM_SHARED((NP, OUT), jnp.float32),
        pltpu.VMEM((EPW_PAD,), jnp.int32),
        pltpu.VMEM((2, CH), jnp.int32),
        pltpu.VMEM((2, CH), jnp.float32),
        pltpu.VMEM((2, CH, OUT), jnp.float32),
        pltpu.VMEM((2, CH), jnp.int32),
        pltpu.SemaphoreType.DMA,
        pltpu.SemaphoreType.DMA,
        pltpu.SemaphoreType.DMA,
    ],
)(_sc_body)


def _pad_per_worker(x, fill):
    x2 = x.reshape(NW, EPW)
    return jnp.pad(x2, ((0, 0), (0, EPW_PAD - EPW)), constant_values=fill).reshape(-1)


def kernel(feat, edge_index, etypes, ntypes, attention, weight, m_bias,
           loop_weight, h_bias):
    w_all = jnp.concatenate([weight, loop_weight], axis=0)
    b_all = jnp.concatenate([m_bias, h_bias[:, 0, :]], axis=0)
    b_all = b_all.reshape(J, 1, OUT)

    srcp = _pad_per_worker(edge_index[0], 0).reshape(NW * NCH, CH)
    etp = _pad_per_worker(etypes, 0).reshape(NW * NCH, CH)
    dstp = _pad_per_worker(edge_index[1], 0).reshape(NW * NCH, CH)
    attp = _pad_per_worker(attention.reshape(E), 0.0)
    ntp = jnp.pad(ntypes, (0, NP - N))

    ERB = NW * NCH // NBLK
    table, idxg = pl.pallas_call(
        _mm_body,
        grid=(NBLK,),
        in_specs=[
            pl.BlockSpec((BN, IN), lambda b: (b, 0)),
            pl.BlockSpec((J, IN, OUT), lambda b: (0, 0, 0)),
            pl.BlockSpec((J, 1, OUT), lambda b: (0, 0, 0)),
            pl.BlockSpec((ERB, CH), lambda b: (b, 0)),
            pl.BlockSpec((ERB, CH), lambda b: (b, 0)),
        ],
        out_specs=[
            pl.BlockSpec((J, BN, OUT), lambda b: (0, b, 0)),
            pl.BlockSpec((ERB, CH), lambda b: (b, 0)),
        ],
        out_shape=[
            jax.ShapeDtypeStruct((J, N, OUT), jnp.float32),
            jax.ShapeDtypeStruct((NW * NCH, CH), jnp.int32),
        ],
    )(feat, w_all, b_all, srcp, etp)
    table_flat = table.reshape(J * N, OUT)
    idxg = idxg.reshape(-1)

    parts = _sc_kernel(table_flat, idxg, dstp, attp, ntp)

    out = pl.pallas_call(
        _add_body,
        grid=(NBLK,),
        in_specs=[
            pl.BlockSpec((1, BN, OUT), lambda b: (0, b, 0)),
            pl.BlockSpec((1, BN, OUT), lambda b: (1, b, 0)),
        ],
        out_specs=pl.BlockSpec((BN, OUT), lambda b: (b, 0)),
        out_shape=jax.ShapeDtypeStruct((N, OUT), jnp.float32),
    )(parts, parts)
    return out.reshape(N, 1, OUT)


# speedup vs baseline: 14.6172x; 1.0001x over previous
import functools

import jax
import jax.numpy as jnp
from jax import lax
from jax.experimental import pallas as pl
from jax.experimental.pallas import tpu as pltpu
from jax.experimental.pallas import tpu_sc as plsc

N = 10000
E = 320000
IN = 128
OUT = 128
R = 8
T = 4
J = R + T

NC = 2
NS = 16
NW = NC * NS

CH = 128
EPW = E // NW
NCH = 80
EPW_PAD = NCH * CH

NP = 10240
ROWS_PER_TILE = NP // NS
INIT_CH = ROWS_PER_TILE // CH

BN = 1000
NBLK = N // BN


def _mm_body(feat_ref, w_ref, b_ref, s_ref, e_ref, table_ref, idx_ref):
    for j in range(J):
        table_ref[j] = (
            jnp.dot(feat_ref[...], w_ref[j], preferred_element_type=jnp.float32)
            + b_ref[j, 0]
        )
    idx_ref[...] = e_ref[...] * N + s_ref[...]


def _add_body(a_ref, b_ref, out_ref):
    out_ref[...] = a_ref[0] + b_ref[0]


def _sc_body(table_hbm, idx_hbm, dst_hbm, att_hbm, nt_hbm, parts_hbm,
             acc, idx_all, dst_v, att_v, rows_v, nt_v,
             gsem0, gsem1, msem):
    c = lax.axis_index("c")
    s = lax.axis_index("s")
    row_base = s * ROWS_PER_TILE
    wid = s * NC + c
    ebase = wid * EPW_PAD

    meta = pltpu.async_copy(idx_hbm.at[pl.ds(ebase, EPW_PAD)], idx_all, msem)

    @pl.when(c == 0)
    def _init_loop():
        def ibody(k, _):
            gb = row_base + k * CH
            pltpu.sync_copy(nt_hbm.at[pl.ds(gb, CH)], nt_v.at[0])
            for q in range(CH // 16):
                nt16 = nt_v[0, pl.ds(q * 16, 16)]
                rows16 = gb + q * 16 + lax.iota(jnp.int32, 16)
                nt_v[1, pl.ds(q * 16, 16)] = (nt16 + R) * N + rows16
            pltpu.async_copy(table_hbm.at[nt_v.at[1]], rows_v.at[0], gsem0).wait()
            pltpu.sync_copy(rows_v.at[0], acc.at[pl.ds(gb, CH)])
            return 0

        lax.fori_loop(0, INIT_CH, ibody, 0)

    @pl.when(c == 1)
    def _init_zero():
        def zbody(r, _):
            for q in range(OUT // 16):
                rows_v[0, r, pl.ds(q * 16, 16)] = jnp.zeros((16,), jnp.float32)
            return 0

        lax.fori_loop(0, CH, zbody, 0)
        for k in range(INIT_CH):
            pltpu.sync_copy(rows_v.at[0], acc.at[pl.ds(row_base + k * CH, CH)])

    meta.wait()

    HC = CH // 4

    def start_chunk(ci, slot, sem):
        for h in range(4):
            pltpu.async_copy(
                table_hbm.at[idx_all.at[pl.ds(ci * CH + h * HC, HC)]],
                rows_v.at[slot, pl.ds(h * HC, HC)], sem
            )
        pltpu.async_copy(att_hbm.at[pl.ds(ebase + ci * CH, CH)],
                         att_v.at[slot], sem)
        pltpu.async_copy(dst_hbm.at[wid * NCH + ci], dst_v.at[slot], sem)

    def wait_chunk(slot, sem):
        pltpu.make_async_copy(
            table_hbm.at[idx_all.at[pl.ds(0, CH)]], rows_v.at[slot], sem
        ).wait()
        pltpu.make_async_copy(att_hbm.at[pl.ds(0, CH)], att_v.at[slot],
                              sem).wait()
        pltpu.make_async_copy(dst_hbm.at[0], dst_v.at[slot], sem).wait()

    def scale_scatter(ci, slot):
        def sbody(g, _):
            att16 = att_v[slot, pl.ds(g * 16, 16)]
            for j in range(16):
                e = g * 16 + j
                spl = jnp.full((16,), att16[j])
                for q in range(OUT // 16):
                    sl = pl.ds(q * 16, 16)
                    rows_v[slot, e, sl] = rows_v[slot, e, sl] * spl
            return 0

        lax.fori_loop(0, CH // 16, sbody, 0)
        pltpu.sync_copy(rows_v.at[slot], acc.at[dst_v.at[slot]], add=True)

    start_chunk(0, 0, gsem0)
    start_chunk(1, 1, gsem1)

    plsc.subcore_barrier()

    def pair(t, _):
        ca = 2 * t
        wait_chunk(0, gsem0)
        scale_scatter(ca, 0)

        @pl.when(ca + 2 < NCH)
        def _():
            start_chunk(ca + 2, 0, gsem0)

        wait_chunk(1, gsem1)
        scale_scatter(ca + 1, 1)

        @pl.when(ca + 3 < NCH)
        def _():
            start_chunk(ca + 3, 1, gsem1)

        return 0

    lax.fori_loop(0, NCH // 2, pair, 0)

    plsc.subcore_barrier()
    for k in range(INIT_CH):
        rb = row_base + k * CH
        pltpu.sync_copy(acc.at[pl.ds(rb, CH)], rows_v.at[0])
        pltpu.sync_copy(rows_v.at[0], parts_hbm.at[c, pl.ds(rb, CH)])


_sc_kernel = functools.partial(
    pl.kernel,
    out_type=jax.ShapeDtypeStruct((NC, NP, OUT), jnp.float32),
    mesh=plsc.VectorSubcoreMesh(
        core_axis_name="c", subcore_axis_name="s", num_cores=NC, num_subcores=NS
    ),
    scratch_types=[
        pltpu.VMEM_SHARED((NP, OUT), jnp.float32),
        pltpu.VMEM((EPW_PAD,), jnp.int32),
        pltpu.VMEM((2, CH), jnp.int32),
        pltpu.VMEM((2, CH), jnp.float32),
        pltpu.VMEM((2, CH, OUT), jnp.float32),
        pltpu.VMEM((2, CH), jnp.int32),
        pltpu.SemaphoreType.DMA,
        pltpu.SemaphoreType.DMA,
        pltpu.SemaphoreType.DMA,
    ],
)(_sc_body)


def _pad_per_worker(x, fill):
    x2 = x.reshape(NW, EPW)
    return jnp.pad(x2, ((0, 0), (0, EPW_PAD - EPW)), constant_values=fill).reshape(-1)


def kernel(feat, edge_index, etypes, ntypes, attention, weight, m_bias,
           loop_weight, h_bias):
    w_all = jnp.concatenate([weight, loop_weight], axis=0)
    b_all = jnp.concatenate([m_bias, h_bias[:, 0, :]], axis=0)
    b_all = b_all.reshape(J, 1, OUT)

    srcp = _pad_per_worker(edge_index[0], 0).reshape(NW * NCH, CH)
    etp = _pad_per_worker(etypes, 0).reshape(NW * NCH, CH)
    dstp = _pad_per_worker(edge_index[1], 0).reshape(NW * NCH, CH)
    attp = _pad_per_worker(attention.reshape(E), 0.0)
    ntp = jnp.pad(ntypes, (0, NP - N))

    ERB = NW * NCH // NBLK
    table, idxg = pl.pallas_call(
        _mm_body,
        grid=(NBLK,),
        in_specs=[
            pl.BlockSpec((BN, IN), lambda b: (b, 0)),
            pl.BlockSpec((J, IN, OUT), lambda b: (0, 0, 0)),
            pl.BlockSpec((J, 1, OUT), lambda b: (0, 0, 0)),
            pl.BlockSpec((ERB, CH), lambda b: (b, 0)),
            pl.BlockSpec((ERB, CH), lambda b: (b, 0)),
        ],
        out_specs=[
            pl.BlockSpec((J, BN, OUT), lambda b: (0, b, 0)),
            pl.BlockSpec((ERB, CH), lambda b: (b, 0)),
        ],
        out_shape=[
            jax.ShapeDtypeStruct((J, N, OUT), jnp.float32),
            jax.ShapeDtypeStruct((NW * NCH, CH), jnp.int32),
        ],
    )(feat, w_all, b_all, srcp, etp)
    table_flat = table.reshape(J * N, OUT)
    idxg = idxg.reshape(-1)

    parts = _sc_kernel(table_flat, idxg, dstp, attp, ntp)

    out = pl.pallas_call(
        _add_body,
        grid=(NBLK,),
        in_specs=[
            pl.BlockSpec((1, BN, OUT), lambda b: (0, b, 0)),
            pl.BlockSpec((1, BN, OUT), lambda b: (1, b, 0)),
        ],
        out_specs=pl.BlockSpec((BN, OUT), lambda b: (b, 0)),
        out_shape=jax.ShapeDtypeStruct((N, OUT), jnp.float32),
    )(parts, parts)
    return out.reshape(N, 1, OUT)
